# baseline (speedup 1.0000x reference)
import functools

import jax
import jax.numpy as jnp
from jax import lax
from jax.experimental import pallas as pl
from jax.experimental.pallas import tpu as pltpu
from jax.experimental.pallas import tpu_sc as plsc

_NC = 2
_NS = 16
_NW = _NC * _NS


@functools.cache
def _sc_gather(start: int, count: int, n_table: int):
    bpw = count // _NW

    @functools.partial(
        pl.kernel,
        mesh=plsc.VectorSubcoreMesh(core_axis_name="c", subcore_axis_name="s"),
        out_type=jax.ShapeDtypeStruct((count,), jnp.float32),
        scratch_types=[
            pltpu.VMEM((bpw,), jnp.int32),
            pltpu.VMEM((bpw,), jnp.float32),
            pltpu.SemaphoreType.DMA,
        ],
    )
    def gather(w_hbm, idx_hbm, out_hbm, idx_v, out_v, sem):
        wid = lax.axis_index("s") * _NC + lax.axis_index("c")
        base = wid * bpw
        pltpu.sync_copy(idx_hbm.at[pl.ds(start + base, bpw)], idx_v)
        pltpu.async_copy(w_hbm.at[idx_v], out_v, sem).wait()
        pltpu.sync_copy(out_v, out_hbm.at[pl.ds(base, bpw)])

    return gather


def _mul_body(scale_ref, e_ref, *rest):
    o_ref = rest[-1]
    i = pl.program_id(0)
    for r in range(e_ref.shape[0]):
        o_ref[r] = e_ref[r] * scale_ref[0, i, r]


def _tc_scale_mul_part(scale3d, e3d, prev_out, blk_off: int, block_b: int):
    b, h, w = e3d.shape
    nb = scale3d.shape[1]
    in_specs = [
        pl.BlockSpec((1, nb, block_b), lambda i: (0, 0, 0),
                     memory_space=pltpu.SMEM),
        pl.BlockSpec((block_b, h, w), lambda i: (i + blk_off, 0, 0)),
    ]
    args = [scale3d, e3d]
    kwargs = {}
    if prev_out is not None:
        in_specs.append(pl.BlockSpec(memory_space=pl.ANY))
        args.append(prev_out)
        kwargs["input_output_aliases"] = {2: 0}
    return pl.pallas_call(
        _mul_body,
        grid=(nb,),
        in_specs=in_specs,
        out_specs=pl.BlockSpec((block_b, h, w), lambda i: (i + blk_off, 0, 0)),
        out_shape=jax.ShapeDtypeStruct((b, h, w), e3d.dtype),
        compiler_params=pltpu.CompilerParams(
            dimension_semantics=("arbitrary",),
        ),
        **kwargs,
    )(*args)


def kernel(E_in, idx, weights):
    b = E_in.shape[0]
    block_b = 128
    split = 2 * block_b
    nt = weights.shape[0]
    idx_flat = idx.reshape(b).astype(jnp.int32)
    s_lo = _sc_gather(0, split, nt)(weights, idx_flat)
    s_hi = _sc_gather(split, b - split, nt)(weights, idx_flat)
    out0 = _tc_scale_mul_part(
        s_lo.reshape(1, split // block_b, block_b), E_in, None, 0, block_b)
    out = _tc_scale_mul_part(
        s_hi.reshape(1, (b - split) // block_b, block_b), E_in, out0,
        split // block_b, block_b)
    return out


# speedup vs baseline: 1.0112x; 1.0112x over previous
import functools

import jax
import jax.numpy as jnp
from jax import lax
from jax.experimental import pallas as pl
from jax.experimental.pallas import tpu as pltpu
from jax.experimental.pallas import tpu_sc as plsc

_NC = 2
_NS = 16
_NW = _NC * _NS


@functools.cache
def _sc_gather(start: int, count: int, n_table: int):
    nw = _NS
    bpw = count // nw

    @functools.partial(
        pl.kernel,
        mesh=plsc.VectorSubcoreMesh(core_axis_name="c", subcore_axis_name="s",
                                    num_cores=1),
        out_type=jax.ShapeDtypeStruct((count,), jnp.float32),
        scratch_types=[
            pltpu.VMEM((bpw,), jnp.int32),
            pltpu.VMEM((bpw,), jnp.float32),
            pltpu.SemaphoreType.DMA,
        ],
    )
    def gather(w_hbm, idx_hbm, out_hbm, idx_v, out_v, sem):
        wid = lax.axis_index("s")
        base = wid * bpw
        pltpu.sync_copy(idx_hbm.at[pl.ds(start + base, bpw)], idx_v)
        pltpu.async_copy(w_hbm.at[idx_v], out_v, sem).wait()
        pltpu.sync_copy(out_v, out_hbm.at[pl.ds(base, bpw)])

    return gather


def _mul_body(scale_ref, e_ref, *rest):
    o_ref = rest[-1]
    i = pl.program_id(0)
    for r in range(e_ref.shape[0]):
        o_ref[r] = e_ref[r] * scale_ref[0, i, r]


def _tc_scale_mul_part(scale3d, e3d, prev_out, blk_off: int, block_b: int):
    b, h, w = e3d.shape
    nb = scale3d.shape[1]
    in_specs = [
        pl.BlockSpec((1, nb, block_b), lambda i: (0, 0, 0),
                     memory_space=pltpu.SMEM),
        pl.BlockSpec((block_b, h, w), lambda i: (i + blk_off, 0, 0)),
    ]
    args = [scale3d, e3d]
    kwargs = {}
    if prev_out is not None:
        in_specs.append(pl.BlockSpec(memory_space=pl.ANY))
        args.append(prev_out)
        kwargs["input_output_aliases"] = {2: 0}
    return pl.pallas_call(
        _mul_body,
        grid=(nb,),
        in_specs=in_specs,
        out_specs=pl.BlockSpec((block_b, h, w), lambda i: (i + blk_off, 0, 0)),
        out_shape=jax.ShapeDtypeStruct((b, h, w), e3d.dtype),
        compiler_params=pltpu.CompilerParams(
            dimension_semantics=("arbitrary",),
        ),
        **kwargs,
    )(*args)


def kernel(E_in, idx, weights):
    b = E_in.shape[0]
    block_b = 128
    split = 2 * block_b
    nt = weights.shape[0]
    idx_flat = idx.reshape(b).astype(jnp.int32)
    s_lo = _sc_gather(0, split, nt)(weights, idx_flat)
    s_hi = _sc_gather(split, b - split, nt)(weights, idx_flat)
    out0 = _tc_scale_mul_part(
        s_lo.reshape(1, split // block_b, block_b), E_in, None, 0, block_b)
    out = _tc_scale_mul_part(
        s_hi.reshape(1, (b - split) // block_b, block_b), E_in, out0,
        split // block_b, block_b)
    return out
